# baseline (speedup 1.0000x reference)
import functools

import jax
import jax.numpy as jnp
from jax import lax
from jax.experimental import pallas as pl
from jax.experimental.pallas import tpu as pltpu, tpu_sc as plsc

_B = 4096
_S = 32
_F = 128
_NW = 32
_BW = _B // _NW
_NBUF = 4
_N = 10000


def _sc_body(feat_hbm, fbf_hbm, nodes_hbm, neighT_hbm, loc_hbm,
             self_out, neigh_out,
             idx_s, nodes_v, loc_v, self_buf, acc_sh, tbl_sh,
             bufs, gsem, ssem, selfsem):
    c = lax.axis_index("c")
    q = lax.axis_index("s")
    w = c * 16 + q
    base = w * _BW
    lbase = q * _BW

    pltpu.sync_copy(neighT_hbm.at[w], idx_s)
    pltpu.sync_copy(loc_hbm.at[pl.ds(base, _BW)], loc_v)

    pltpu.sync_copy(nodes_hbm.at[pl.ds(base, _BW)], nodes_v)
    pltpu.async_copy(feat_hbm.at[nodes_v], self_buf, selfsem)

    rpt = _N // 16
    pltpu.sync_copy(fbf_hbm.at[pl.ds(q * rpt, rpt)],
                    tbl_sh.at[pl.ds(q * rpt, rpt)])
    plsc.subcore_barrier()

    for b in range(_NBUF):
        pltpu.async_copy(tbl_sh.at[idx_s.at[b]], bufs[b], gsem[b])

    def wait_gather(b):
        pltpu.make_async_copy(tbl_sh.at[pl.ds(0, _BW)], bufs[b],
                              gsem[b]).wait()

    def wait_scatter(b):
        pltpu.make_async_copy(bufs[b], acc_sh.at[pl.ds(lbase, _BW)],
                              ssem[b]).wait()

    def do_round(j, b, refill_j):
        wait_gather(b)
        pltpu.async_copy(bufs[b], acc_sh.at[loc_v], ssem[b], add=True)
        if refill_j is not None:
            wait_scatter(b)
            pltpu.async_copy(tbl_sh.at[idx_s.at[refill_j]], bufs[b],
                             gsem[b])

    wait_gather(0)
    pltpu.sync_copy(bufs[0], acc_sh.at[loc_v])
    pltpu.async_copy(tbl_sh.at[idx_s.at[_NBUF]], bufs[0], gsem[0])

    for j in range(1, _NBUF):
        do_round(j, j, j + _NBUF)

    def group(g, carry):
        for b in range(_NBUF):
            j = g * _NBUF + b
            do_round(j, b, j + _NBUF)
        return carry

    lax.fori_loop(1, _S // _NBUF - 1, group, 0)

    for j in range(_S - _NBUF, _S):
        do_round(j, j % _NBUF, None)
    for b in range(_NBUF):
        wait_scatter(b)

    pltpu.make_async_copy(feat_hbm.at[pl.ds(0, _BW)], self_buf,
                          selfsem).wait()
    pltpu.sync_copy(self_buf, self_out.at[pl.ds(base, _BW)])
    pltpu.sync_copy(acc_sh.at[pl.ds(lbase, _BW)],
                    neigh_out.at[pl.ds(base, _BW)])


def _sc_gather(features, fbf, nodes, neighTw, loc):
    mesh = plsc.VectorSubcoreMesh(core_axis_name="c", subcore_axis_name="s")
    f32 = jnp.float32
    bf16 = jnp.bfloat16
    return pl.kernel(
        _sc_body,
        out_type=(jax.ShapeDtypeStruct((_B, _F), f32),
                  jax.ShapeDtypeStruct((_B, _F), bf16)),
        mesh=mesh,
        compiler_params=pltpu.CompilerParams(use_tc_tiling_on_sc=False),
        scratch_types=[
            pltpu.VMEM((_S, _BW), jnp.int32),
            pltpu.VMEM((_BW,), jnp.int32),
            pltpu.VMEM((_BW,), jnp.int32),
            pltpu.VMEM((_BW, _F), f32),
            pltpu.VMEM_SHARED((_B // 2, _F), bf16),
            pltpu.VMEM_SHARED((_N, _F), bf16),
            [pltpu.VMEM((_BW, _F), bf16) for _ in range(_NBUF)],
            [pltpu.SemaphoreType.DMA for _ in range(_NBUF)],
            [pltpu.SemaphoreType.DMA for _ in range(_NBUF)],
            pltpu.SemaphoreType.DMA,
        ],
    )(features, fbf, nodes, neighTw, loc)


def _tc_body(self_ref, neigh_ref, w_ref, out_ref):
    w1 = w_ref[0:_F, :]
    w2 = w_ref[_F:2 * _F, :] * (1.0 / _S)
    a = lax.dot_general(w1, self_ref[...], (((0,), (1,)), ((), ())),
                        preferred_element_type=jnp.float32)
    b = lax.dot_general(w2, neigh_ref[...].astype(jnp.float32),
                        (((0,), (1,)), ((), ())),
                        preferred_element_type=jnp.float32)
    out_ref[...] = jnp.maximum(a + b, 0.0)


def _tc_project(self_feats, neigh_sum, weight):
    blk = 1024
    grid = (_B // blk,)
    return pl.pallas_call(
        _tc_body,
        grid=grid,
        in_specs=[
            pl.BlockSpec((blk, _F), lambda i: (i, 0)),
            pl.BlockSpec((blk, _F), lambda i: (i, 0)),
            pl.BlockSpec((2 * _F, _F), lambda i: (0, 0)),
        ],
        out_specs=pl.BlockSpec((_F, blk), lambda i: (0, i)),
        out_shape=jax.ShapeDtypeStruct((_F, _B), jnp.float32),
    )(self_feats, neigh_sum, weight)


@jax.jit
def kernel(nodes, neigh_idx, features, weight):
    nodes = nodes.astype(jnp.int32)
    neighTw = jnp.transpose(
        neigh_idx.astype(jnp.int32).reshape(_NW, _BW, _S), (0, 2, 1))
    loc = jnp.arange(_B, dtype=jnp.int32) % (_B // 2)
    fbf = features.astype(jnp.bfloat16)
    self_feats, neigh_sum = _sc_gather(features, fbf, nodes, neighTw, loc)
    return _tc_project(self_feats, neigh_sum, weight)


# speedup vs baseline: 1.1888x; 1.1888x over previous
import functools

import jax
import jax.numpy as jnp
from jax import lax
from jax.experimental import pallas as pl
from jax.experimental.pallas import tpu as pltpu, tpu_sc as plsc

_B = 4096
_S = 32
_F = 128
_NW = 32
_BW = _B // _NW
_NBUF = 4


def _sc_body(feat_hbm, nodes_hbm, neighT_hbm, loc_hbm,
             self_out, neigh_out,
             idx_s, nodes_v, loc_v, self_buf, acc_sh,
             bufs, gsem, ssem, selfsem):
    c = lax.axis_index("c")
    q = lax.axis_index("s")
    w = c * 16 + q
    base = w * _BW
    lbase = q * _BW

    pltpu.sync_copy(neighT_hbm.at[w], idx_s)
    pltpu.sync_copy(loc_hbm.at[pl.ds(base, _BW)], loc_v)

    pltpu.sync_copy(nodes_hbm.at[pl.ds(base, _BW)], nodes_v)
    pltpu.async_copy(feat_hbm.at[nodes_v], self_buf, selfsem)

    for b in range(_NBUF):
        pltpu.async_copy(feat_hbm.at[idx_s.at[b]], bufs[b], gsem[b])

    def wait_gather(b):
        pltpu.make_async_copy(feat_hbm.at[pl.ds(0, _BW)], bufs[b],
                              gsem[b]).wait()

    def wait_scatter(b):
        pltpu.make_async_copy(bufs[b], acc_sh.at[pl.ds(lbase, _BW)],
                              ssem[b]).wait()

    def do_round(j, b, refill_j):
        wait_gather(b)
        pltpu.async_copy(bufs[b], acc_sh.at[loc_v], ssem[b], add=True)
        if refill_j is not None:
            wait_scatter(b)
            pltpu.async_copy(feat_hbm.at[idx_s.at[refill_j]], bufs[b],
                             gsem[b])

    wait_gather(0)
    pltpu.sync_copy(bufs[0], acc_sh.at[loc_v])
    pltpu.async_copy(feat_hbm.at[idx_s.at[_NBUF]], bufs[0], gsem[0])

    for j in range(1, _NBUF):
        do_round(j, j, j + _NBUF)

    def group(g, carry):
        for b in range(_NBUF):
            j = g * _NBUF + b
            do_round(j, b, j + _NBUF)
        return carry

    lax.fori_loop(1, _S // _NBUF - 1, group, 0)

    for j in range(_S - _NBUF, _S):
        do_round(j, j % _NBUF, None)
    for b in range(_NBUF):
        wait_scatter(b)

    pltpu.make_async_copy(feat_hbm.at[pl.ds(0, _BW)], self_buf,
                          selfsem).wait()
    pltpu.sync_copy(self_buf, self_out.at[pl.ds(base, _BW)])
    pltpu.sync_copy(acc_sh.at[pl.ds(lbase, _BW)],
                    neigh_out.at[pl.ds(base, _BW)])


def _sc_gather(features, nodes, neighTw, loc):
    mesh = plsc.VectorSubcoreMesh(core_axis_name="c", subcore_axis_name="s")
    f32 = jnp.float32
    return pl.kernel(
        _sc_body,
        out_type=(jax.ShapeDtypeStruct((_B, _F), f32),
                  jax.ShapeDtypeStruct((_B, _F), f32)),
        mesh=mesh,
        compiler_params=pltpu.CompilerParams(use_tc_tiling_on_sc=False),
        scratch_types=[
            pltpu.VMEM((_S, _BW), jnp.int32),
            pltpu.VMEM((_BW,), jnp.int32),
            pltpu.VMEM((_BW,), jnp.int32),
            pltpu.VMEM((_BW, _F), f32),
            pltpu.VMEM_SHARED((_B // 2, _F), f32),
            [pltpu.VMEM((_BW, _F), f32) for _ in range(_NBUF)],
            [pltpu.SemaphoreType.DMA for _ in range(_NBUF)],
            [pltpu.SemaphoreType.DMA for _ in range(_NBUF)],
            pltpu.SemaphoreType.DMA,
        ],
    )(features, nodes, neighTw, loc)


def _tc_body(self_ref, neigh_ref, w_ref, out_ref):
    w1 = w_ref[0:_F, :]
    w2 = w_ref[_F:2 * _F, :] * (1.0 / _S)
    a = lax.dot_general(w1, self_ref[...], (((0,), (1,)), ((), ())),
                        preferred_element_type=jnp.float32)
    b = lax.dot_general(w2, neigh_ref[...], (((0,), (1,)), ((), ())),
                        preferred_element_type=jnp.float32)
    out_ref[...] = jnp.maximum(a + b, 0.0)


def _tc_project(self_feats, neigh_sum, weight):
    blk = 1024
    grid = (_B // blk,)
    return pl.pallas_call(
        _tc_body,
        grid=grid,
        in_specs=[
            pl.BlockSpec((blk, _F), lambda i: (i, 0)),
            pl.BlockSpec((blk, _F), lambda i: (i, 0)),
            pl.BlockSpec((2 * _F, _F), lambda i: (0, 0)),
        ],
        out_specs=pl.BlockSpec((_F, blk), lambda i: (0, i)),
        out_shape=jax.ShapeDtypeStruct((_F, _B), jnp.float32),
    )(self_feats, neigh_sum, weight)


@jax.jit
def kernel(nodes, neigh_idx, features, weight):
    nodes = nodes.astype(jnp.int32)
    neighTw = jnp.transpose(
        neigh_idx.astype(jnp.int32).reshape(_NW, _BW, _S), (0, 2, 1))
    loc = jnp.arange(_B, dtype=jnp.int32) % (_B // 2)
    self_feats, neigh_sum = _sc_gather(features, nodes, neighTw, loc)
    return _tc_project(self_feats, neigh_sum, weight)
